# Initial kernel scaffold; baseline (speedup 1.0000x reference)
import jax
import jax.numpy as jnp
from jax.experimental import pallas as pl


def kernel(x, edge_index, edge_attr, batch, charge, W_ce, b_ce, W_ae, b_ae, W_be, b_be, W_eu1, b_eu1, W_eu2, b_eu2, W_nu1, b_nu1, W_nu2, b_nu2, gamma, beta, W_p1, b_p1, W_p2, b_p2, W_p3, b_p3):
    raise NotImplementedError("write your pallas kernel here")


# speedup vs baseline: 1.4808x; 1.4808x over previous
import functools

import jax
import jax.numpy as jnp
from jax import lax
from jax.experimental import pallas as pl
from jax.experimental.pallas import tpu as pltpu
from jax.experimental.pallas import tpu_sc as plsc

_EPS = 1e-5
_SCALE = 1.0 / (1.0 + _EPS) ** 0.5

_CH = 128
_NW = 32


def _softplus(v):
    return jnp.maximum(v, 0.0) + jnp.log1p(jnp.exp(-jnp.abs(v)))


def _pre_body(x_ref, b_ref, ch_ref, wce_ref, bce_ref, waet_ref, waeb_ref,
              bae_ref, h0_ref):
    cf = jnp.dot(ch_ref[...], wce_ref[...],
                 preferred_element_type=jnp.float32) + bce_ref[...]
    q = jnp.dot(cf, waeb_ref[...], preferred_element_type=jnp.float32)
    g = q.shape[0]
    oh = (b_ref[0] == lax.broadcasted_iota(jnp.int32, (1, g), 1)
          ).astype(jnp.float32)
    h0_ref[...] = (jnp.dot(x_ref[...], waet_ref[...],
                           preferred_element_type=jnp.float32)
                   + jnp.dot(oh, q, preferred_element_type=jnp.float32)
                   + bae_ref[...])


def _edge_body_first(xr_ref, xc_ref, ea_ref, wbe_ref, bbe_ref, w1_ref, b1_ref,
                     w2_ref, b2_ref, wn1_ref, bn1_ref, wn2_ref, bn2_ref,
                     enew_ref, msg_ref):
    e = jnp.dot(ea_ref[...], wbe_ref[...],
                preferred_element_type=jnp.float32) + bbe_ref[...]
    _edge_core(xr_ref[...], xc_ref[...], e, w1_ref, b1_ref, w2_ref, b2_ref,
               wn1_ref, bn1_ref, wn2_ref, bn2_ref, enew_ref, msg_ref)


def _edge_body(xr_ref, xc_ref, e_ref, w1_ref, b1_ref, w2_ref, b2_ref,
               wn1_ref, bn1_ref, wn2_ref, bn2_ref, enew_ref, msg_ref):
    _edge_core(xr_ref[...], xc_ref[...], e_ref[...], w1_ref, b1_ref, w2_ref,
               b2_ref, wn1_ref, bn1_ref, wn2_ref, bn2_ref, enew_ref, msg_ref)


def _edge_core(xr, xc, e, w1_ref, b1_ref, w2_ref, b2_ref, wn1_ref, bn1_ref,
               wn2_ref, bn2_ref, enew_ref, msg_ref):
    ein = jnp.concatenate([xr, xc, e], axis=1)
    t = _softplus(jnp.dot(ein, w1_ref[...],
                          preferred_element_type=jnp.float32) + b1_ref[...])
    enew = jnp.dot(t, w2_ref[...],
                   preferred_element_type=jnp.float32) + b2_ref[...]
    nin = jnp.concatenate([xr, enew], axis=1)
    t2 = _softplus(jnp.dot(nin, wn1_ref[...],
                           preferred_element_type=jnp.float32) + bn1_ref[...])
    msg = jnp.dot(t2, wn2_ref[...],
                  preferred_element_type=jnp.float32) + bn2_ref[...]
    enew_ref[...] = enew
    msg_ref[...] = msg


def _update_body(acc_ref, h_ref, g_ref, be_ref, out_ref):
    s = acc_ref[0] + acc_ref[1]
    xn = s * _SCALE * g_ref[...] + be_ref[...]
    out_ref[...] = _softplus(xn) + h_ref[...]


def _final_body(acc_ref, h_ref, g_ref, be_ref, b_ref, wp1_ref, bp1_ref,
                wp2_ref, bp2_ref, wp3_ref, bp3_ref, out_ref):
    s = acc_ref[0] + acc_ref[1]
    h = _softplus(s * _SCALE * g_ref[...] + be_ref[...]) + h_ref[...]
    g = out_ref.shape[0]
    oh = (b_ref[0] == lax.broadcasted_iota(jnp.int32, (g, 1), 0)
          ).astype(jnp.float32)
    sums = jnp.dot(oh, h, preferred_element_type=jnp.float32)
    counts = jnp.sum(oh, axis=1, keepdims=True)
    graph = sums / jnp.maximum(counts, 1.0)
    z = _softplus(jnp.dot(graph, wp1_ref[...],
                          preferred_element_type=jnp.float32) + bp1_ref[...])
    z = _softplus(jnp.dot(z, wp2_ref[...],
                          preferred_element_type=jnp.float32) + bp2_ref[...])
    out_ref[...] = jnp.dot(z, wp3_ref[...],
                           preferred_element_type=jnp.float32) + bp3_ref[...]


def _make_sc_gather(n, d, e_pad):
    cpt = e_pad // _CH // _NW
    mesh = plsc.VectorSubcoreMesh(core_axis_name="c", subcore_axis_name="s")

    def body(h_hbm, row2_hbm, col2_hbm, xr_hbm, xc_hbm,
             idxr_v, idxc_v, bufr, bufc, semr, semc):
        c = lax.axis_index("c")
        s = lax.axis_index("s")
        w = s * 2 + c
        start = pl.multiple_of(w * cpt, 8)
        pltpu.sync_copy(row2_hbm.at[pl.ds(start, cpt)], idxr_v)
        pltpu.sync_copy(col2_hbm.at[pl.ds(start, cpt)], idxc_v)

        def chunk(j, carry):
            gr = pltpu.async_copy(h_hbm.at[idxr_v.at[j]], bufr, semr)
            gc = pltpu.async_copy(h_hbm.at[idxc_v.at[j]], bufc, semc)
            gr.wait()
            gc.wait()
            pltpu.sync_copy(bufr, xr_hbm.at[pl.ds((start + j) * _CH, _CH)])
            pltpu.sync_copy(bufc, xc_hbm.at[pl.ds((start + j) * _CH, _CH)])
            return carry

        lax.fori_loop(0, cpt, chunk, 0)

    return pl.kernel(
        body,
        out_type=(jax.ShapeDtypeStruct((e_pad, d), jnp.float32),
                  jax.ShapeDtypeStruct((e_pad, d), jnp.float32)),
        mesh=mesh,
        compiler_params=pltpu.CompilerParams(use_tc_tiling_on_sc=False),
        scratch_types=[
            pltpu.VMEM((cpt, _CH), jnp.int32),
            pltpu.VMEM((cpt, _CH), jnp.int32),
            pltpu.VMEM((_CH, d), jnp.float32),
            pltpu.VMEM((_CH, d), jnp.float32),
            pltpu.SemaphoreType.DMA,
            pltpu.SemaphoreType.DMA,
        ],
    )


def _make_sc_scatter(n, d, e_pad):
    cpt = e_pad // _CH // _NW
    n_acc = n + 8
    rpt_hi = 632
    rpt_lo = n - 15 * rpt_hi
    mesh = plsc.VectorSubcoreMesh(core_axis_name="c", subcore_axis_name="s")

    def body(msg_hbm, col2_hbm, zeros_hbm, out_hbm, idx_v, mbuf, accs):
        c = lax.axis_index("c")
        s = lax.axis_index("s")
        w = s * 2 + c

        @pl.when(s == 0)
        def _():
            pltpu.sync_copy(zeros_hbm, accs)

        start = pl.multiple_of(w * cpt, 8)
        pltpu.sync_copy(col2_hbm.at[pl.ds(start, cpt)], idx_v)
        plsc.subcore_barrier()

        def chunk(j, carry):
            pltpu.sync_copy(msg_hbm.at[pl.ds((start + j) * _CH, _CH)], mbuf)
            pltpu.sync_copy(mbuf, accs.at[idx_v.at[j]], add=True)
            return carry

        lax.fori_loop(0, cpt, chunk, 0)
        plsc.subcore_barrier()

        @pl.when(s < 15)
        def _():
            pltpu.sync_copy(accs.at[pl.ds(s * rpt_hi, rpt_hi)],
                            out_hbm.at[c, pl.ds(s * rpt_hi, rpt_hi)])

        @pl.when(s == 15)
        def _():
            pltpu.sync_copy(accs.at[pl.ds(15 * rpt_hi, rpt_lo)],
                            out_hbm.at[c, pl.ds(15 * rpt_hi, rpt_lo)])

    return pl.kernel(
        body,
        out_type=jax.ShapeDtypeStruct((2, n, d), jnp.float32),
        mesh=mesh,
        compiler_params=pltpu.CompilerParams(use_tc_tiling_on_sc=False),
        scratch_types=[
            pltpu.VMEM((cpt, _CH), jnp.int32),
            pltpu.VMEM((_CH, d), jnp.float32),
            pltpu.VMEM_SHARED((n_acc, d), jnp.float32),
        ],
    )


def kernel(x, edge_index, edge_attr, batch, charge, W_ce, b_ce, W_ae, b_ae,
           W_be, b_be, W_eu1, b_eu1, W_eu2, b_eu2, W_nu1, b_nu1, W_nu2, b_nu2,
           gamma, beta, W_p1, b_p1, W_p2, b_p2, W_p3, b_p3):
    n, fa = x.shape
    e_num, fb = edge_attr.shape
    g = charge.shape[0]
    d = W_ae.shape[1]
    nl = W_eu1.shape[0]
    h_dim = W_p1.shape[1]

    quantum = _NW * _CH * 8
    e_pad = ((e_num + quantum - 1) // quantum) * quantum
    pad = e_pad - e_num
    row = jnp.concatenate([edge_index[0], jnp.zeros((pad,), jnp.int32)])
    col = edge_index[1]
    colg = jnp.concatenate([col, jnp.zeros((pad,), jnp.int32)])
    cols = jnp.concatenate([col, jnp.full((pad,), n, jnp.int32)])
    row2 = row.reshape(e_pad // _CH, _CH)
    colg2 = colg.reshape(e_pad // _CH, _CH)
    cols2 = cols.reshape(e_pad // _CH, _CH)
    ea_pad = jnp.concatenate(
        [edge_attr, jnp.zeros((pad, fb), jnp.float32)], axis=0)

    bn = 2000
    nb = n // bn
    batch_col = batch.reshape(nb, bn, 1)
    batch_row = batch.reshape(1, 1, n)
    zeros = jnp.zeros((n + 8, d), jnp.float32)

    full = lambda shape: pl.BlockSpec(shape, lambda i: (0,) * len(shape))

    h = pl.pallas_call(
        _pre_body,
        grid=(nb,),
        in_specs=[
            pl.BlockSpec((bn, fa), lambda i: (i, 0)),
            pl.BlockSpec((1, bn, 1), lambda i: (i, 0, 0)),
            full((g, 1)), full((1, W_ce.shape[1])), full((1, W_ce.shape[1])),
            full((fa, d)), full((W_ce.shape[1], d)), full((1, d)),
        ],
        out_specs=pl.BlockSpec((bn, d), lambda i: (i, 0)),
        out_shape=jax.ShapeDtypeStruct((n, d), jnp.float32),
    )(x, batch_col, charge.reshape(g, 1), W_ce, b_ce.reshape(1, -1),
      W_ae[:fa], W_ae[fa:], b_ae.reshape(1, d))

    sc_gather = _make_sc_gather(n, d, e_pad)
    sc_scatter = _make_sc_scatter(n, d, e_pad)

    be_blk = 2560
    eb = e_pad // be_blk

    def edge_mlp(first, xr, xc, e_in, wi):
        w1, b1, w2, b2, wn1, bn1, wn2, bn2 = wi
        f_in = e_in.shape[1]
        in_specs = [
            pl.BlockSpec((be_blk, d), lambda i: (i, 0)),
            pl.BlockSpec((be_blk, d), lambda i: (i, 0)),
            pl.BlockSpec((be_blk, f_in), lambda i: (i, 0)),
        ]
        args = [xr, xc, e_in]
        if first:
            in_specs += [full((fb, d)), full((1, d))]
            args += [W_be, b_be.reshape(1, d)]
        in_specs += [full((3 * d, d)), full((1, d)), full((d, d)),
                     full((1, d)), full((2 * d, d)), full((1, d)),
                     full((d, d)), full((1, d))]
        args += [w1, b1.reshape(1, d), w2, b2.reshape(1, d),
                 wn1, bn1.reshape(1, d), wn2, bn2.reshape(1, d)]
        return pl.pallas_call(
            _edge_body_first if first else _edge_body,
            grid=(eb,),
            in_specs=in_specs,
            out_specs=(pl.BlockSpec((be_blk, d), lambda i: (i, 0)),
                       pl.BlockSpec((be_blk, d), lambda i: (i, 0))),
            out_shape=(jax.ShapeDtypeStruct((e_pad, d), jnp.float32),
                       jax.ShapeDtypeStruct((e_pad, d), jnp.float32)),
        )(*args)

    e = ea_pad
    acc = None
    for i in range(nl):
        xr, xc = sc_gather(h, row2, colg2)
        wi = (W_eu1[i], b_eu1[i], W_eu2[i], b_eu2[i],
              W_nu1[i], b_nu1[i], W_nu2[i], b_nu2[i])
        e, msg = edge_mlp(i == 0, xr, xc, e, wi)
        acc = sc_scatter(msg, cols2, zeros)
        if i < nl - 1:
            h = pl.pallas_call(
                _update_body,
                grid=(nb,),
                in_specs=[
                    pl.BlockSpec((2, bn, d), lambda i: (0, i, 0)),
                    pl.BlockSpec((bn, d), lambda i: (i, 0)),
                    full((1, d)), full((1, d)),
                ],
                out_specs=pl.BlockSpec((bn, d), lambda i: (i, 0)),
                out_shape=jax.ShapeDtypeStruct((n, d), jnp.float32),
            )(acc, h, gamma[i].reshape(1, d), beta[i].reshape(1, d))

    out = pl.pallas_call(
        _final_body,
        grid=(1,),
        in_specs=[
            full((2, n, d)), full((n, d)), full((1, d)), full((1, d)),
            full((1, 1, n)), full((d, h_dim)), full((1, h_dim)),
            full((h_dim, h_dim)), full((1, h_dim)), full((h_dim, 1)),
            full((1, 1)),
        ],
        out_specs=pl.BlockSpec((g, 1), lambda i: (0, 0)),
        out_shape=jax.ShapeDtypeStruct((g, 1), jnp.float32),
    )(acc, h, gamma[nl - 1].reshape(1, d), beta[nl - 1].reshape(1, d),
      batch_row, W_p1, b_p1.reshape(1, h_dim), W_p2, b_p2.reshape(1, h_dim),
      W_p3, b_p3.reshape(1, 1))
    return out[:, 0]


# speedup vs baseline: 1.6130x; 1.0892x over previous
import functools

import jax
import jax.numpy as jnp
from jax import lax
from jax.experimental import pallas as pl
from jax.experimental.pallas import tpu as pltpu
from jax.experimental.pallas import tpu_sc as plsc

_EPS = 1e-5
_SCALE = 1.0 / (1.0 + _EPS) ** 0.5

_CH = 128
_NW = 32


def _softplus(v):
    return jnp.maximum(v, 0.0) + jnp.log1p(jnp.exp(-jnp.abs(v)))


def _pre_body(x_ref, b_ref, ch_ref, wce_ref, bce_ref, waet_ref, waeb_ref,
              bae_ref, h0_ref):
    cf = jnp.dot(ch_ref[...], wce_ref[...],
                 preferred_element_type=jnp.float32) + bce_ref[...]
    q = jnp.dot(cf, waeb_ref[...], preferred_element_type=jnp.float32)
    g = q.shape[0]
    oh = (b_ref[0] == lax.broadcasted_iota(jnp.int32, (1, g), 1)
          ).astype(jnp.float32)
    h0_ref[...] = (jnp.dot(x_ref[...], waet_ref[...],
                           preferred_element_type=jnp.float32)
                   + jnp.dot(oh, q, preferred_element_type=jnp.float32)
                   + bae_ref[...])


def _edge_body_first(xr_ref, xc_ref, ea_ref, wbe_ref, bbe_ref, w1_ref, b1_ref,
                     w2_ref, b2_ref, wn1_ref, bn1_ref, wn2_ref, bn2_ref,
                     enew_ref, msg_ref):
    e = jnp.dot(ea_ref[...], wbe_ref[...],
                preferred_element_type=jnp.float32) + bbe_ref[...]
    _edge_core(xr_ref[...], xc_ref[...], e, w1_ref, b1_ref, w2_ref, b2_ref,
               wn1_ref, bn1_ref, wn2_ref, bn2_ref, enew_ref, msg_ref)


def _edge_body(xr_ref, xc_ref, e_ref, w1_ref, b1_ref, w2_ref, b2_ref,
               wn1_ref, bn1_ref, wn2_ref, bn2_ref, enew_ref, msg_ref):
    _edge_core(xr_ref[...], xc_ref[...], e_ref[...], w1_ref, b1_ref, w2_ref,
               b2_ref, wn1_ref, bn1_ref, wn2_ref, bn2_ref, enew_ref, msg_ref)


def _edge_core(xr, xc, e, w1_ref, b1_ref, w2_ref, b2_ref, wn1_ref, bn1_ref,
               wn2_ref, bn2_ref, enew_ref, msg_ref):
    ein = jnp.concatenate([xr, xc, e], axis=1)
    t = _softplus(jnp.dot(ein, w1_ref[...],
                          preferred_element_type=jnp.float32) + b1_ref[...])
    enew = jnp.dot(t, w2_ref[...],
                   preferred_element_type=jnp.float32) + b2_ref[...]
    nin = jnp.concatenate([xr, enew], axis=1)
    t2 = _softplus(jnp.dot(nin, wn1_ref[...],
                           preferred_element_type=jnp.float32) + bn1_ref[...])
    msg = jnp.dot(t2, wn2_ref[...],
                  preferred_element_type=jnp.float32) + bn2_ref[...]
    enew_ref[...] = enew
    msg_ref[...] = msg


def _update_body(acc_ref, h_ref, g_ref, be_ref, out_ref):
    s = acc_ref[0] + acc_ref[1]
    xn = s * _SCALE * g_ref[...] + be_ref[...]
    out_ref[...] = _softplus(xn) + h_ref[...]


def _final_body(acc_ref, h_ref, g_ref, be_ref, b_ref, wp1_ref, bp1_ref,
                wp2_ref, bp2_ref, wp3_ref, bp3_ref, out_ref):
    s = acc_ref[0] + acc_ref[1]
    h = _softplus(s * _SCALE * g_ref[...] + be_ref[...]) + h_ref[...]
    g = out_ref.shape[0]
    oh = (b_ref[0] == lax.broadcasted_iota(jnp.int32, (g, 1), 0)
          ).astype(jnp.float32)
    sums = jnp.dot(oh, h, preferred_element_type=jnp.float32)
    counts = jnp.sum(oh, axis=1, keepdims=True)
    graph = sums / jnp.maximum(counts, 1.0)
    z = _softplus(jnp.dot(graph, wp1_ref[...],
                          preferred_element_type=jnp.float32) + bp1_ref[...])
    z = _softplus(jnp.dot(z, wp2_ref[...],
                          preferred_element_type=jnp.float32) + bp2_ref[...])
    out_ref[...] = jnp.dot(z, wp3_ref[...],
                           preferred_element_type=jnp.float32) + bp3_ref[...]


_NBUF = 5


def _make_sc_gather(n, d, e_pad):
    cpt = e_pad // _CH // _NW
    nbuf = _NBUF
    steps = cpt // nbuf
    mesh = plsc.VectorSubcoreMesh(core_axis_name="c", subcore_axis_name="s")

    def body(h_hbm, row2_hbm, col2_hbm, xr_hbm, xc_hbm, *scr):
        idxr_v, idxc_v, bufr, bufc = scr[:4]
        gsr = scr[4:4 + nbuf]
        gsc = scr[4 + nbuf:4 + 2 * nbuf]
        wsr = scr[4 + 2 * nbuf:4 + 3 * nbuf]
        wsc = scr[4 + 3 * nbuf:4 + 4 * nbuf]
        c = lax.axis_index("c")
        s = lax.axis_index("s")
        w = s * 2 + c
        start = pl.multiple_of(w * cpt, 8)
        pltpu.sync_copy(row2_hbm.at[pl.ds(start, cpt)], idxr_v)
        pltpu.sync_copy(col2_hbm.at[pl.ds(start, cpt)], idxc_v)

        def sstep(S, carry):
            grs = []
            gcs = []
            for b in range(nbuf):
                @pl.when(S > 0)
                def _(b=b):
                    pltpu.make_async_copy(xr_hbm.at[pl.ds(0, _CH)],
                                          bufr.at[b], wsr[b]).wait()
                    pltpu.make_async_copy(xc_hbm.at[pl.ds(0, _CH)],
                                          bufc.at[b], wsc[b]).wait()
                j = S * nbuf + b
                grs.append(pltpu.async_copy(
                    h_hbm.at[idxr_v.at[j]], bufr.at[b], gsr[b]))
                gcs.append(pltpu.async_copy(
                    h_hbm.at[idxc_v.at[j]], bufc.at[b], gsc[b]))
            for b in range(nbuf):
                j = S * nbuf + b
                grs[b].wait()
                gcs[b].wait()
                pltpu.async_copy(bufr.at[b],
                                 xr_hbm.at[pl.ds((start + j) * _CH, _CH)],
                                 wsr[b])
                pltpu.async_copy(bufc.at[b],
                                 xc_hbm.at[pl.ds((start + j) * _CH, _CH)],
                                 wsc[b])
            return carry

        lax.fori_loop(0, steps, sstep, 0)
        for b in range(nbuf):
            pltpu.make_async_copy(xr_hbm.at[pl.ds(0, _CH)], bufr.at[b],
                                  wsr[b]).wait()
            pltpu.make_async_copy(xc_hbm.at[pl.ds(0, _CH)], bufc.at[b],
                                  wsc[b]).wait()

    return pl.kernel(
        body,
        out_type=(jax.ShapeDtypeStruct((e_pad, d), jnp.float32),
                  jax.ShapeDtypeStruct((e_pad, d), jnp.float32)),
        mesh=mesh,
        compiler_params=pltpu.CompilerParams(use_tc_tiling_on_sc=False),
        scratch_types=[
            pltpu.VMEM((cpt, _CH), jnp.int32),
            pltpu.VMEM((cpt, _CH), jnp.int32),
            pltpu.VMEM((nbuf, _CH, d), jnp.float32),
            pltpu.VMEM((nbuf, _CH, d), jnp.float32),
        ] + [pltpu.SemaphoreType.DMA] * (4 * nbuf),
    )


def _make_sc_scatter(n, d, e_pad):
    cpt = e_pad // _CH // _NW
    n_acc = n + 8
    rpt_hi = 632
    rpt_lo = n - 15 * rpt_hi
    mesh = plsc.VectorSubcoreMesh(core_axis_name="c", subcore_axis_name="s")

    nbuf = _NBUF
    steps = cpt // nbuf

    def body(msg_hbm, col2_hbm, zeros_hbm, out_hbm, *scr):
        idx_v, mbuf, accs = scr[:3]
        rsem = scr[3:3 + nbuf]
        ssem = scr[3 + nbuf:3 + 2 * nbuf]
        c = lax.axis_index("c")
        s = lax.axis_index("s")
        w = s * 2 + c

        @pl.when(s == 0)
        def _():
            pltpu.sync_copy(zeros_hbm, accs)

        start = pl.multiple_of(w * cpt, 8)
        pltpu.sync_copy(col2_hbm.at[pl.ds(start, cpt)], idx_v)
        plsc.subcore_barrier()

        def sstep(S, carry):
            rds = []
            for b in range(nbuf):
                @pl.when(S > 0)
                def _(b=b):
                    pltpu.make_async_copy(msg_hbm.at[pl.ds(0, _CH)],
                                          mbuf.at[b], ssem[b]).wait()
                j = S * nbuf + b
                rds.append(pltpu.async_copy(
                    msg_hbm.at[pl.ds((start + j) * _CH, _CH)], mbuf.at[b],
                    rsem[b]))
            for b in range(nbuf):
                j = S * nbuf + b
                rds[b].wait()
                pltpu.async_copy(mbuf.at[b], accs.at[idx_v.at[j]], ssem[b],
                                 add=True)
            return carry

        lax.fori_loop(0, steps, sstep, 0)
        for b in range(nbuf):
            pltpu.make_async_copy(msg_hbm.at[pl.ds(0, _CH)], mbuf.at[b],
                                  ssem[b]).wait()
        plsc.subcore_barrier()

        @pl.when(s < 15)
        def _():
            pltpu.sync_copy(accs.at[pl.ds(s * rpt_hi, rpt_hi)],
                            out_hbm.at[c, pl.ds(s * rpt_hi, rpt_hi)])

        @pl.when(s == 15)
        def _():
            pltpu.sync_copy(accs.at[pl.ds(15 * rpt_hi, rpt_lo)],
                            out_hbm.at[c, pl.ds(15 * rpt_hi, rpt_lo)])

    return pl.kernel(
        body,
        out_type=jax.ShapeDtypeStruct((2, n, d), jnp.float32),
        mesh=mesh,
        compiler_params=pltpu.CompilerParams(use_tc_tiling_on_sc=False),
        scratch_types=[
            pltpu.VMEM((cpt, _CH), jnp.int32),
            pltpu.VMEM((_NBUF, _CH, d), jnp.float32),
            pltpu.VMEM_SHARED((n_acc, d), jnp.float32),
        ] + [pltpu.SemaphoreType.DMA] * (2 * _NBUF),
    )


def kernel(x, edge_index, edge_attr, batch, charge, W_ce, b_ce, W_ae, b_ae,
           W_be, b_be, W_eu1, b_eu1, W_eu2, b_eu2, W_nu1, b_nu1, W_nu2, b_nu2,
           gamma, beta, W_p1, b_p1, W_p2, b_p2, W_p3, b_p3):
    n, fa = x.shape
    e_num, fb = edge_attr.shape
    g = charge.shape[0]
    d = W_ae.shape[1]
    nl = W_eu1.shape[0]
    h_dim = W_p1.shape[1]

    quantum = _NW * _CH * 8
    e_pad = ((e_num + quantum - 1) // quantum) * quantum
    pad = e_pad - e_num
    row = jnp.concatenate([edge_index[0], jnp.zeros((pad,), jnp.int32)])
    col = edge_index[1]
    colg = jnp.concatenate([col, jnp.zeros((pad,), jnp.int32)])
    cols = jnp.concatenate([col, jnp.full((pad,), n, jnp.int32)])
    row2 = row.reshape(e_pad // _CH, _CH)
    colg2 = colg.reshape(e_pad // _CH, _CH)
    cols2 = cols.reshape(e_pad // _CH, _CH)
    ea_pad = jnp.concatenate(
        [edge_attr, jnp.zeros((pad, fb), jnp.float32)], axis=0)

    bn = 2000
    nb = n // bn
    batch_col = batch.reshape(nb, bn, 1)
    batch_row = batch.reshape(1, 1, n)
    zeros = jnp.zeros((n + 8, d), jnp.float32)

    full = lambda shape: pl.BlockSpec(shape, lambda i: (0,) * len(shape))

    h = pl.pallas_call(
        _pre_body,
        grid=(nb,),
        in_specs=[
            pl.BlockSpec((bn, fa), lambda i: (i, 0)),
            pl.BlockSpec((1, bn, 1), lambda i: (i, 0, 0)),
            full((g, 1)), full((1, W_ce.shape[1])), full((1, W_ce.shape[1])),
            full((fa, d)), full((W_ce.shape[1], d)), full((1, d)),
        ],
        out_specs=pl.BlockSpec((bn, d), lambda i: (i, 0)),
        out_shape=jax.ShapeDtypeStruct((n, d), jnp.float32),
    )(x, batch_col, charge.reshape(g, 1), W_ce, b_ce.reshape(1, -1),
      W_ae[:fa], W_ae[fa:], b_ae.reshape(1, d))

    sc_gather = _make_sc_gather(n, d, e_pad)
    sc_scatter = _make_sc_scatter(n, d, e_pad)

    be_blk = 2560
    eb = e_pad // be_blk

    def edge_mlp(first, xr, xc, e_in, wi):
        w1, b1, w2, b2, wn1, bn1, wn2, bn2 = wi
        f_in = e_in.shape[1]
        in_specs = [
            pl.BlockSpec((be_blk, d), lambda i: (i, 0)),
            pl.BlockSpec((be_blk, d), lambda i: (i, 0)),
            pl.BlockSpec((be_blk, f_in), lambda i: (i, 0)),
        ]
        args = [xr, xc, e_in]
        if first:
            in_specs += [full((fb, d)), full((1, d))]
            args += [W_be, b_be.reshape(1, d)]
        in_specs += [full((3 * d, d)), full((1, d)), full((d, d)),
                     full((1, d)), full((2 * d, d)), full((1, d)),
                     full((d, d)), full((1, d))]
        args += [w1, b1.reshape(1, d), w2, b2.reshape(1, d),
                 wn1, bn1.reshape(1, d), wn2, bn2.reshape(1, d)]
        return pl.pallas_call(
            _edge_body_first if first else _edge_body,
            grid=(eb,),
            in_specs=in_specs,
            out_specs=(pl.BlockSpec((be_blk, d), lambda i: (i, 0)),
                       pl.BlockSpec((be_blk, d), lambda i: (i, 0))),
            out_shape=(jax.ShapeDtypeStruct((e_pad, d), jnp.float32),
                       jax.ShapeDtypeStruct((e_pad, d), jnp.float32)),
        )(*args)

    e = ea_pad
    acc = None
    for i in range(nl):
        xr, xc = sc_gather(h, row2, colg2)
        wi = (W_eu1[i], b_eu1[i], W_eu2[i], b_eu2[i],
              W_nu1[i], b_nu1[i], W_nu2[i], b_nu2[i])
        e, msg = edge_mlp(i == 0, xr, xc, e, wi)
        acc = sc_scatter(msg, cols2, zeros)
        if i < nl - 1:
            h = pl.pallas_call(
                _update_body,
                grid=(nb,),
                in_specs=[
                    pl.BlockSpec((2, bn, d), lambda i: (0, i, 0)),
                    pl.BlockSpec((bn, d), lambda i: (i, 0)),
                    full((1, d)), full((1, d)),
                ],
                out_specs=pl.BlockSpec((bn, d), lambda i: (i, 0)),
                out_shape=jax.ShapeDtypeStruct((n, d), jnp.float32),
            )(acc, h, gamma[i].reshape(1, d), beta[i].reshape(1, d))

    out = pl.pallas_call(
        _final_body,
        grid=(1,),
        in_specs=[
            full((2, n, d)), full((n, d)), full((1, d)), full((1, d)),
            full((1, 1, n)), full((d, h_dim)), full((1, h_dim)),
            full((h_dim, h_dim)), full((1, h_dim)), full((h_dim, 1)),
            full((1, 1)),
        ],
        out_specs=pl.BlockSpec((g, 1), lambda i: (0, 0)),
        out_shape=jax.ShapeDtypeStruct((g, 1), jnp.float32),
    )(acc, h, gamma[nl - 1].reshape(1, d), beta[nl - 1].reshape(1, d),
      batch_row, W_p1, b_p1.reshape(1, h_dim), W_p2, b_p2.reshape(1, h_dim),
      W_p3, b_p3.reshape(1, 1))
    return out[:, 0]


# speedup vs baseline: 1.9891x; 1.2332x over previous
import functools

import jax
import jax.numpy as jnp
from jax import lax
from jax.experimental import pallas as pl
from jax.experimental.pallas import tpu as pltpu
from jax.experimental.pallas import tpu_sc as plsc

_EPS = 1e-5
_SCALE = 1.0 / (1.0 + _EPS) ** 0.5

_CH = 128
_NW = 32


def _softplus(v):
    return jnp.maximum(v, 0.0) + jnp.log1p(jnp.exp(-jnp.abs(v)))


def _pack_cols(h16):
    u = jax.lax.bitcast_convert_type(h16, jnp.uint16)
    lo = u[:, :32].astype(jnp.uint32)
    hi = u[:, 32:].astype(jnp.uint32)
    return jax.lax.bitcast_convert_type(lo | (hi << 16), jnp.int32)


def _unpack_cols(v):
    lo = jax.lax.bitcast_convert_type(v << 16, jnp.float32)
    hi = jax.lax.bitcast_convert_type(v & jnp.int32(-65536), jnp.float32)
    return jnp.concatenate([lo, hi], axis=1)


def _pre_body(x_ref, b_ref, ch_ref, wce_ref, bce_ref, waet_ref, waeb_ref,
              bae_ref, h0_ref, h16_ref):
    cf = jnp.dot(ch_ref[...], wce_ref[...],
                 preferred_element_type=jnp.float32) + bce_ref[...]
    q = jnp.dot(cf, waeb_ref[...], preferred_element_type=jnp.float32)
    g = q.shape[0]
    oh = (b_ref[0] == lax.broadcasted_iota(jnp.int32, (1, g), 1)
          ).astype(jnp.float32)
    h0 = (jnp.dot(x_ref[...], waet_ref[...],
                  preferred_element_type=jnp.float32)
          + jnp.dot(oh, q, preferred_element_type=jnp.float32)
          + bae_ref[...])
    h0_ref[...] = h0
    h16_ref[...] = h0.astype(jnp.bfloat16)


def _edge_body_first(xr_ref, xc_ref, ea_ref, wbe_ref, bbe_ref, w1_ref, b1_ref,
                     w2_ref, b2_ref, wn1_ref, bn1_ref, wn2_ref, bn2_ref,
                     enew_ref, msg_ref):
    e = jnp.dot(ea_ref[...], wbe_ref[...],
                preferred_element_type=jnp.float32) + bbe_ref[...]
    _edge_core(_unpack_cols(xr_ref[...]), _unpack_cols(xc_ref[...]), e,
               w1_ref, b1_ref, w2_ref, b2_ref, wn1_ref, bn1_ref, wn2_ref,
               bn2_ref, enew_ref, msg_ref)


def _edge_body(xr_ref, xc_ref, e_ref, w1_ref, b1_ref, w2_ref, b2_ref,
               wn1_ref, bn1_ref, wn2_ref, bn2_ref, enew_ref, msg_ref):
    _edge_core(_unpack_cols(xr_ref[...]), _unpack_cols(xc_ref[...]),
               e_ref[...], w1_ref, b1_ref, w2_ref, b2_ref, wn1_ref, bn1_ref,
               wn2_ref, bn2_ref, enew_ref, msg_ref)


def _edge_core(xr, xc, e, w1_ref, b1_ref, w2_ref, b2_ref, wn1_ref, bn1_ref,
               wn2_ref, bn2_ref, enew_ref, msg_ref):
    ein = jnp.concatenate([xr, xc, e], axis=1)
    t = _softplus(jnp.dot(ein, w1_ref[...],
                          preferred_element_type=jnp.float32) + b1_ref[...])
    enew = jnp.dot(t, w2_ref[...],
                   preferred_element_type=jnp.float32) + b2_ref[...]
    nin = jnp.concatenate([xr, enew], axis=1)
    t2 = _softplus(jnp.dot(nin, wn1_ref[...],
                           preferred_element_type=jnp.float32) + bn1_ref[...])
    msg = jnp.dot(t2, wn2_ref[...],
                  preferred_element_type=jnp.float32) + bn2_ref[...]
    enew_ref[...] = enew
    msg_ref[...] = msg


def _update_body(acc_ref, h_ref, g_ref, be_ref, out_ref, out16_ref):
    s = acc_ref[0] + acc_ref[1]
    xn = s * _SCALE * g_ref[...] + be_ref[...]
    h = _softplus(xn) + h_ref[...]
    out_ref[...] = h
    out16_ref[...] = h.astype(jnp.bfloat16)


def _final_body(acc_ref, h_ref, g_ref, be_ref, b_ref, wp1_ref, bp1_ref,
                wp2_ref, bp2_ref, wp3_ref, bp3_ref, out_ref):
    s = acc_ref[0] + acc_ref[1]
    h = _softplus(s * _SCALE * g_ref[...] + be_ref[...]) + h_ref[...]
    g = out_ref.shape[0]
    oh = (b_ref[0] == lax.broadcasted_iota(jnp.int32, (g, 1), 0)
          ).astype(jnp.float32)
    sums = jnp.dot(oh, h, preferred_element_type=jnp.float32)
    counts = jnp.sum(oh, axis=1, keepdims=True)
    graph = sums / jnp.maximum(counts, 1.0)
    z = _softplus(jnp.dot(graph, wp1_ref[...],
                          preferred_element_type=jnp.float32) + bp1_ref[...])
    z = _softplus(jnp.dot(z, wp2_ref[...],
                          preferred_element_type=jnp.float32) + bp2_ref[...])
    out_ref[...] = jnp.dot(z, wp3_ref[...],
                           preferred_element_type=jnp.float32) + bp3_ref[...]


_NBUF = 5


def _make_sc_gather(n, dw, e_pad):
    cpt = e_pad // _CH // _NW
    nbuf = _NBUF
    steps = cpt // nbuf
    mesh = plsc.VectorSubcoreMesh(core_axis_name="c", subcore_axis_name="s")

    def body(h_hbm, row2_hbm, col2_hbm, xr_hbm, xc_hbm, *scr):
        idxr_v, idxc_v, bufr, bufc = scr[:4]
        gsr = scr[4:4 + nbuf]
        gsc = scr[4 + nbuf:4 + 2 * nbuf]
        wsr = scr[4 + 2 * nbuf:4 + 3 * nbuf]
        wsc = scr[4 + 3 * nbuf:4 + 4 * nbuf]
        c = lax.axis_index("c")
        s = lax.axis_index("s")
        w = s * 2 + c
        start = pl.multiple_of(w * cpt, 8)
        pltpu.sync_copy(row2_hbm.at[pl.ds(start, cpt)], idxr_v)
        pltpu.sync_copy(col2_hbm.at[pl.ds(start, cpt)], idxc_v)

        def sstep(S, carry):
            grs = []
            gcs = []
            for b in range(nbuf):
                @pl.when(S > 0)
                def _(b=b):
                    pltpu.make_async_copy(xr_hbm.at[pl.ds(0, _CH)],
                                          bufr.at[b], wsr[b]).wait()
                    pltpu.make_async_copy(xc_hbm.at[pl.ds(0, _CH)],
                                          bufc.at[b], wsc[b]).wait()
                j = S * nbuf + b
                grs.append(pltpu.async_copy(
                    h_hbm.at[idxr_v.at[j]], bufr.at[b], gsr[b]))
                gcs.append(pltpu.async_copy(
                    h_hbm.at[idxc_v.at[j]], bufc.at[b], gsc[b]))
            for b in range(nbuf):
                j = S * nbuf + b
                grs[b].wait()
                gcs[b].wait()
                pltpu.async_copy(bufr.at[b],
                                 xr_hbm.at[pl.ds((start + j) * _CH, _CH)],
                                 wsr[b])
                pltpu.async_copy(bufc.at[b],
                                 xc_hbm.at[pl.ds((start + j) * _CH, _CH)],
                                 wsc[b])
            return carry

        lax.fori_loop(0, steps, sstep, 0)
        for b in range(nbuf):
            pltpu.make_async_copy(xr_hbm.at[pl.ds(0, _CH)], bufr.at[b],
                                  wsr[b]).wait()
            pltpu.make_async_copy(xc_hbm.at[pl.ds(0, _CH)], bufc.at[b],
                                  wsc[b]).wait()

    return pl.kernel(
        body,
        out_type=(jax.ShapeDtypeStruct((e_pad, dw), jnp.int32),
                  jax.ShapeDtypeStruct((e_pad, dw), jnp.int32)),
        mesh=mesh,
        compiler_params=pltpu.CompilerParams(use_tc_tiling_on_sc=False),
        scratch_types=[
            pltpu.VMEM((cpt, _CH), jnp.int32),
            pltpu.VMEM((cpt, _CH), jnp.int32),
            pltpu.VMEM((nbuf, _CH, dw), jnp.int32),
            pltpu.VMEM((nbuf, _CH, dw), jnp.int32),
        ] + [pltpu.SemaphoreType.DMA] * (4 * nbuf),
    )


def _make_sc_scatter(n, d, e_pad):
    cpt = e_pad // _CH // _NW
    n_acc = n + 8
    rpt_hi = 632
    rpt_lo = n - 15 * rpt_hi
    mesh = plsc.VectorSubcoreMesh(core_axis_name="c", subcore_axis_name="s")

    nbuf = _NBUF
    steps = cpt // nbuf

    def body(msg_hbm, col2_hbm, zeros_hbm, out_hbm, *scr):
        idx_v, mbuf, accs = scr[:3]
        rsem = scr[3:3 + nbuf]
        ssem = scr[3 + nbuf:3 + 2 * nbuf]
        c = lax.axis_index("c")
        s = lax.axis_index("s")
        w = s * 2 + c

        @pl.when(s == 0)
        def _():
            pltpu.sync_copy(zeros_hbm, accs)

        start = pl.multiple_of(w * cpt, 8)
        pltpu.sync_copy(col2_hbm.at[pl.ds(start, cpt)], idx_v)
        plsc.subcore_barrier()

        def sstep(S, carry):
            rds = []
            for b in range(nbuf):
                @pl.when(S > 0)
                def _(b=b):
                    pltpu.make_async_copy(msg_hbm.at[pl.ds(0, _CH)],
                                          mbuf.at[b], ssem[b]).wait()
                j = S * nbuf + b
                rds.append(pltpu.async_copy(
                    msg_hbm.at[pl.ds((start + j) * _CH, _CH)], mbuf.at[b],
                    rsem[b]))
            for b in range(nbuf):
                j = S * nbuf + b
                rds[b].wait()
                pltpu.async_copy(mbuf.at[b], accs.at[idx_v.at[j]], ssem[b],
                                 add=True)
            return carry

        lax.fori_loop(0, steps, sstep, 0)
        for b in range(nbuf):
            pltpu.make_async_copy(msg_hbm.at[pl.ds(0, _CH)], mbuf.at[b],
                                  ssem[b]).wait()
        plsc.subcore_barrier()

        @pl.when(s < 15)
        def _():
            pltpu.sync_copy(accs.at[pl.ds(s * rpt_hi, rpt_hi)],
                            out_hbm.at[c, pl.ds(s * rpt_hi, rpt_hi)])

        @pl.when(s == 15)
        def _():
            pltpu.sync_copy(accs.at[pl.ds(15 * rpt_hi, rpt_lo)],
                            out_hbm.at[c, pl.ds(15 * rpt_hi, rpt_lo)])

    return pl.kernel(
        body,
        out_type=jax.ShapeDtypeStruct((2, n, d), jnp.float32),
        mesh=mesh,
        compiler_params=pltpu.CompilerParams(use_tc_tiling_on_sc=False),
        scratch_types=[
            pltpu.VMEM((cpt, _CH), jnp.int32),
            pltpu.VMEM((_NBUF, _CH, d), jnp.float32),
            pltpu.VMEM_SHARED((n_acc, d), jnp.float32),
        ] + [pltpu.SemaphoreType.DMA] * (2 * _NBUF),
    )


def kernel(x, edge_index, edge_attr, batch, charge, W_ce, b_ce, W_ae, b_ae,
           W_be, b_be, W_eu1, b_eu1, W_eu2, b_eu2, W_nu1, b_nu1, W_nu2, b_nu2,
           gamma, beta, W_p1, b_p1, W_p2, b_p2, W_p3, b_p3):
    n, fa = x.shape
    e_num, fb = edge_attr.shape
    g = charge.shape[0]
    d = W_ae.shape[1]
    nl = W_eu1.shape[0]
    h_dim = W_p1.shape[1]

    quantum = _NW * _CH * 8
    e_pad = ((e_num + quantum - 1) // quantum) * quantum
    pad = e_pad - e_num
    row = jnp.concatenate([edge_index[0], jnp.zeros((pad,), jnp.int32)])
    col = edge_index[1]
    colg = jnp.concatenate([col, jnp.zeros((pad,), jnp.int32)])
    cols = jnp.concatenate([col, jnp.full((pad,), n, jnp.int32)])
    row2 = row.reshape(e_pad // _CH, _CH)
    colg2 = colg.reshape(e_pad // _CH, _CH)
    cols2 = cols.reshape(e_pad // _CH, _CH)
    ea_pad = jnp.concatenate(
        [edge_attr, jnp.zeros((pad, fb), jnp.float32)], axis=0)

    bn = 2000
    nb = n // bn
    batch_col = batch.reshape(nb, bn, 1)
    batch_row = batch.reshape(1, 1, n)
    zeros = jnp.zeros((n + 8, d), jnp.float32)

    full = lambda shape: pl.BlockSpec(shape, lambda i: (0,) * len(shape))

    h, h16 = pl.pallas_call(
        _pre_body,
        grid=(nb,),
        in_specs=[
            pl.BlockSpec((bn, fa), lambda i: (i, 0)),
            pl.BlockSpec((1, bn, 1), lambda i: (i, 0, 0)),
            full((g, 1)), full((1, W_ce.shape[1])), full((1, W_ce.shape[1])),
            full((fa, d)), full((W_ce.shape[1], d)), full((1, d)),
        ],
        out_specs=(pl.BlockSpec((bn, d), lambda i: (i, 0)),
                   pl.BlockSpec((bn, d), lambda i: (i, 0))),
        out_shape=(jax.ShapeDtypeStruct((n, d), jnp.float32),
                   jax.ShapeDtypeStruct((n, d), jnp.bfloat16)),
    )(x, batch_col, charge.reshape(g, 1), W_ce, b_ce.reshape(1, -1),
      W_ae[:fa], W_ae[fa:], b_ae.reshape(1, d))

    sc_gather = _make_sc_gather(n, d // 2, e_pad)
    sc_scatter = _make_sc_scatter(n, d, e_pad)

    be_blk = 2560
    eb = e_pad // be_blk

    def edge_mlp(first, xr, xc, e_in, wi):
        w1, b1, w2, b2, wn1, bn1, wn2, bn2 = wi
        f_in = e_in.shape[1]
        in_specs = [
            pl.BlockSpec((be_blk, d // 2), lambda i: (i, 0)),
            pl.BlockSpec((be_blk, d // 2), lambda i: (i, 0)),
            pl.BlockSpec((be_blk, f_in), lambda i: (i, 0)),
        ]
        args = [xr, xc, e_in]
        if first:
            in_specs += [full((fb, d)), full((1, d))]
            args += [W_be, b_be.reshape(1, d)]
        in_specs += [full((3 * d, d)), full((1, d)), full((d, d)),
                     full((1, d)), full((2 * d, d)), full((1, d)),
                     full((d, d)), full((1, d))]
        args += [w1, b1.reshape(1, d), w2, b2.reshape(1, d),
                 wn1, bn1.reshape(1, d), wn2, bn2.reshape(1, d)]
        return pl.pallas_call(
            _edge_body_first if first else _edge_body,
            grid=(eb,),
            in_specs=in_specs,
            out_specs=(pl.BlockSpec((be_blk, d), lambda i: (i, 0)),
                       pl.BlockSpec((be_blk, d), lambda i: (i, 0))),
            out_shape=(jax.ShapeDtypeStruct((e_pad, d), jnp.float32),
                       jax.ShapeDtypeStruct((e_pad, d), jnp.float32)),
        )(*args)

    e = ea_pad
    acc = None
    for i in range(nl):
        xr, xc = sc_gather(_pack_cols(h16), row2, colg2)
        wi = (W_eu1[i], b_eu1[i], W_eu2[i], b_eu2[i],
              W_nu1[i], b_nu1[i], W_nu2[i], b_nu2[i])
        e, msg = edge_mlp(i == 0, xr, xc, e, wi)
        acc = sc_scatter(msg, cols2, zeros)
        if i < nl - 1:
            h, h16 = pl.pallas_call(
                _update_body,
                grid=(nb,),
                in_specs=[
                    pl.BlockSpec((2, bn, d), lambda i: (0, i, 0)),
                    pl.BlockSpec((bn, d), lambda i: (i, 0)),
                    full((1, d)), full((1, d)),
                ],
                out_specs=(pl.BlockSpec((bn, d), lambda i: (i, 0)),
                           pl.BlockSpec((bn, d), lambda i: (i, 0))),
                out_shape=(jax.ShapeDtypeStruct((n, d), jnp.float32),
                           jax.ShapeDtypeStruct((n, d), jnp.bfloat16)),
            )(acc, h, gamma[i].reshape(1, d), beta[i].reshape(1, d))

    out = pl.pallas_call(
        _final_body,
        grid=(1,),
        in_specs=[
            full((2, n, d)), full((n, d)), full((1, d)), full((1, d)),
            full((1, 1, n)), full((d, h_dim)), full((1, h_dim)),
            full((h_dim, h_dim)), full((1, h_dim)), full((h_dim, 1)),
            full((1, 1)),
        ],
        out_specs=pl.BlockSpec((g, 1), lambda i: (0, 0)),
        out_shape=jax.ShapeDtypeStruct((g, 1), jnp.float32),
    )(acc, h, gamma[nl - 1].reshape(1, d), beta[nl - 1].reshape(1, d),
      batch_row, W_p1, b_p1.reshape(1, h_dim), W_p2, b_p2.reshape(1, h_dim),
      W_p3, b_p3.reshape(1, 1))
    return out[:, 0]


# speedup vs baseline: 2.1926x; 1.1023x over previous
import functools

import jax
import jax.numpy as jnp
from jax import lax
from jax.experimental import pallas as pl
from jax.experimental.pallas import tpu as pltpu
from jax.experimental.pallas import tpu_sc as plsc

_EPS = 1e-5
_SCALE = 1.0 / (1.0 + _EPS) ** 0.5

_CH = 128
_NW = 32


def _softplus(v):
    return jnp.maximum(v, 0.0) + jnp.log1p(jnp.exp(-jnp.abs(v)))


def _pack_cols(h16):
    u = jax.lax.bitcast_convert_type(h16, jnp.uint16)
    lo = u[:, :32].astype(jnp.uint32)
    hi = u[:, 32:].astype(jnp.uint32)
    return jax.lax.bitcast_convert_type(lo | (hi << 16), jnp.int32)


def _unpack_cols(v):
    lo = jax.lax.bitcast_convert_type(v << 16, jnp.float32)
    hi = jax.lax.bitcast_convert_type(v & jnp.int32(-65536), jnp.float32)
    return jnp.concatenate([lo, hi], axis=1)


def _pre_body(x_ref, b_ref, ch_ref, wce_ref, bce_ref, waet_ref, waeb_ref,
              bae_ref, h0_ref, h16_ref):
    cf = jnp.dot(ch_ref[...], wce_ref[...],
                 preferred_element_type=jnp.float32) + bce_ref[...]
    q = jnp.dot(cf, waeb_ref[...], preferred_element_type=jnp.float32)
    g = q.shape[0]
    oh = (b_ref[0] == lax.broadcasted_iota(jnp.int32, (1, g), 1)
          ).astype(jnp.float32)
    h0 = (jnp.dot(x_ref[...], waet_ref[...],
                  preferred_element_type=jnp.float32)
          + jnp.dot(oh, q, preferred_element_type=jnp.float32)
          + bae_ref[...])
    h0_ref[...] = h0
    h16_ref[...] = h0.astype(jnp.bfloat16)


def _edge_body_first(xr_ref, xc_ref, ea_ref, wbe_ref, bbe_ref, w1_ref, b1_ref,
                     w2_ref, b2_ref, wn1_ref, bn1_ref, wn2_ref, bn2_ref,
                     enew_ref, msg_ref):
    e = jnp.dot(ea_ref[...], wbe_ref[...],
                preferred_element_type=jnp.float32) + bbe_ref[...]
    _edge_core(_unpack_cols(xr_ref[...]), _unpack_cols(xc_ref[...]), e,
               w1_ref, b1_ref, w2_ref, b2_ref, wn1_ref, bn1_ref, wn2_ref,
               bn2_ref, enew_ref, msg_ref)


def _edge_body(xr_ref, xc_ref, e_ref, w1_ref, b1_ref, w2_ref, b2_ref,
               wn1_ref, bn1_ref, wn2_ref, bn2_ref, enew_ref, msg_ref):
    _edge_core(_unpack_cols(xr_ref[...]), _unpack_cols(xc_ref[...]),
               e_ref[...], w1_ref, b1_ref, w2_ref, b2_ref, wn1_ref, bn1_ref,
               wn2_ref, bn2_ref, enew_ref, msg_ref)


def _edge_core(xr, xc, e, w1_ref, b1_ref, w2_ref, b2_ref, wn1_ref, bn1_ref,
               wn2_ref, bn2_ref, enew_ref, msg_ref):
    ein = jnp.concatenate([xr, xc, e], axis=1)
    t = _softplus(jnp.dot(ein, w1_ref[...],
                          preferred_element_type=jnp.float32) + b1_ref[...])
    enew = jnp.dot(t, w2_ref[...],
                   preferred_element_type=jnp.float32) + b2_ref[...]
    nin = jnp.concatenate([xr, enew], axis=1)
    t2 = _softplus(jnp.dot(nin, wn1_ref[...],
                           preferred_element_type=jnp.float32) + bn1_ref[...])
    msg = jnp.dot(t2, wn2_ref[...],
                  preferred_element_type=jnp.float32) + bn2_ref[...]
    enew_ref[...] = enew
    msg_ref[...] = msg


def _update_body(acc_ref, h_ref, g_ref, be_ref, out_ref, out16_ref):
    s = acc_ref[0] + acc_ref[1]
    xn = s * _SCALE * g_ref[...] + be_ref[...]
    h = _softplus(xn) + h_ref[...]
    out_ref[...] = h
    out16_ref[...] = h.astype(jnp.bfloat16)


def _final_body(acc_ref, h_ref, g_ref, be_ref, b_ref, wp1_ref, bp1_ref,
                wp2_ref, bp2_ref, wp3_ref, bp3_ref, out_ref):
    s = acc_ref[0] + acc_ref[1]
    h = _softplus(s * _SCALE * g_ref[...] + be_ref[...]) + h_ref[...]
    g = out_ref.shape[0]
    oh = (b_ref[0] == lax.broadcasted_iota(jnp.int32, (g, 1), 0)
          ).astype(jnp.float32)
    sums = jnp.dot(oh, h, preferred_element_type=jnp.float32)
    counts = jnp.sum(oh, axis=1, keepdims=True)
    graph = sums / jnp.maximum(counts, 1.0)
    z = _softplus(jnp.dot(graph, wp1_ref[...],
                          preferred_element_type=jnp.float32) + bp1_ref[...])
    z = _softplus(jnp.dot(z, wp2_ref[...],
                          preferred_element_type=jnp.float32) + bp2_ref[...])
    out_ref[...] = jnp.dot(z, wp3_ref[...],
                           preferred_element_type=jnp.float32) + bp3_ref[...]


_NBUF = 5


def _make_sc_gather(n, dw, e_half, cbase):
    cpt = e_half // _CH // _NW
    nbuf = _NBUF
    steps = cpt // nbuf
    mesh = plsc.VectorSubcoreMesh(core_axis_name="c", subcore_axis_name="s")

    def body(h_hbm, row2_hbm, col2_hbm, xr_hbm, xc_hbm, *scr):
        idxr_v, idxc_v, bufr, bufc = scr[:4]
        gsr = scr[4:4 + nbuf]
        gsc = scr[4 + nbuf:4 + 2 * nbuf]
        wsr = scr[4 + 2 * nbuf:4 + 3 * nbuf]
        wsc = scr[4 + 3 * nbuf:4 + 4 * nbuf]
        c = lax.axis_index("c")
        s = lax.axis_index("s")
        w = s * 2 + c
        start = pl.multiple_of(w * cpt, 8)
        gstart = pl.multiple_of(cbase + w * cpt, 8)
        pltpu.sync_copy(row2_hbm.at[pl.ds(gstart, cpt)], idxr_v)
        pltpu.sync_copy(col2_hbm.at[pl.ds(gstart, cpt)], idxc_v)

        def sstep(S, carry):
            grs = []
            gcs = []
            for b in range(nbuf):
                @pl.when(S > 0)
                def _(b=b):
                    pltpu.make_async_copy(xr_hbm.at[pl.ds(0, _CH)],
                                          bufr.at[b], wsr[b]).wait()
                    pltpu.make_async_copy(xc_hbm.at[pl.ds(0, _CH)],
                                          bufc.at[b], wsc[b]).wait()
                j = S * nbuf + b
                grs.append(pltpu.async_copy(
                    h_hbm.at[idxr_v.at[j]], bufr.at[b], gsr[b]))
                gcs.append(pltpu.async_copy(
                    h_hbm.at[idxc_v.at[j]], bufc.at[b], gsc[b]))
            for b in range(nbuf):
                j = S * nbuf + b
                grs[b].wait()
                gcs[b].wait()
                pltpu.async_copy(bufr.at[b],
                                 xr_hbm.at[pl.ds((start + j) * _CH, _CH)],
                                 wsr[b])
                pltpu.async_copy(bufc.at[b],
                                 xc_hbm.at[pl.ds((start + j) * _CH, _CH)],
                                 wsc[b])
            return carry

        lax.fori_loop(0, steps, sstep, 0)
        for b in range(nbuf):
            pltpu.make_async_copy(xr_hbm.at[pl.ds(0, _CH)], bufr.at[b],
                                  wsr[b]).wait()
            pltpu.make_async_copy(xc_hbm.at[pl.ds(0, _CH)], bufc.at[b],
                                  wsc[b]).wait()

    return pl.kernel(
        body,
        out_type=(jax.ShapeDtypeStruct((e_half, dw), jnp.int32),
                  jax.ShapeDtypeStruct((e_half, dw), jnp.int32)),
        mesh=mesh,
        compiler_params=pltpu.CompilerParams(use_tc_tiling_on_sc=False),
        scratch_types=[
            pltpu.VMEM((cpt, _CH), jnp.int32),
            pltpu.VMEM((cpt, _CH), jnp.int32),
            pltpu.VMEM((nbuf, _CH, dw), jnp.int32),
            pltpu.VMEM((nbuf, _CH, dw), jnp.int32),
        ] + [pltpu.SemaphoreType.DMA] * (4 * nbuf),
    )


def _make_sc_scatter(n, d, e_half, cbase):
    cpt = e_half // _CH // _NW
    n_acc = n + 8
    rpt_hi = 632
    rpt_lo = n_acc - 15 * rpt_hi
    mesh = plsc.VectorSubcoreMesh(core_axis_name="c", subcore_axis_name="s")

    nbuf = _NBUF
    steps = cpt // nbuf

    def body(msg_hbm, col2_hbm, init_hbm, out_hbm, *scr):
        idx_v, mbuf, accs = scr[:3]
        rsem = scr[3:3 + nbuf]
        ssem = scr[3 + nbuf:3 + 2 * nbuf]
        c = lax.axis_index("c")
        s = lax.axis_index("s")
        w = s * 2 + c

        @pl.when(s == 0)
        def _():
            pltpu.sync_copy(init_hbm.at[c], accs)

        start = pl.multiple_of(w * cpt, 8)
        gstart = pl.multiple_of(cbase + w * cpt, 8)
        pltpu.sync_copy(col2_hbm.at[pl.ds(gstart, cpt)], idx_v)
        plsc.subcore_barrier()

        def sstep(S, carry):
            rds = []
            for b in range(nbuf):
                @pl.when(S > 0)
                def _(b=b):
                    pltpu.make_async_copy(msg_hbm.at[pl.ds(0, _CH)],
                                          mbuf.at[b], ssem[b]).wait()
                j = S * nbuf + b
                rds.append(pltpu.async_copy(
                    msg_hbm.at[pl.ds((start + j) * _CH, _CH)], mbuf.at[b],
                    rsem[b]))
            for b in range(nbuf):
                j = S * nbuf + b
                rds[b].wait()
                pltpu.async_copy(mbuf.at[b], accs.at[idx_v.at[j]], ssem[b],
                                 add=True)
            return carry

        lax.fori_loop(0, steps, sstep, 0)
        for b in range(nbuf):
            pltpu.make_async_copy(msg_hbm.at[pl.ds(0, _CH)], mbuf.at[b],
                                  ssem[b]).wait()
        plsc.subcore_barrier()

        @pl.when(s < 15)
        def _():
            pltpu.sync_copy(accs.at[pl.ds(s * rpt_hi, rpt_hi)],
                            out_hbm.at[c, pl.ds(s * rpt_hi, rpt_hi)])

        @pl.when(s == 15)
        def _():
            pltpu.sync_copy(accs.at[pl.ds(15 * rpt_hi, rpt_lo)],
                            out_hbm.at[c, pl.ds(15 * rpt_hi, rpt_lo)])

    return pl.kernel(
        body,
        out_type=jax.ShapeDtypeStruct((2, n_acc, d), jnp.float32),
        mesh=mesh,
        compiler_params=pltpu.CompilerParams(use_tc_tiling_on_sc=False),
        scratch_types=[
            pltpu.VMEM((cpt, _CH), jnp.int32),
            pltpu.VMEM((_NBUF, _CH, d), jnp.float32),
            pltpu.VMEM_SHARED((n_acc, d), jnp.float32),
        ] + [pltpu.SemaphoreType.DMA] * (2 * _NBUF),
    )


def kernel(x, edge_index, edge_attr, batch, charge, W_ce, b_ce, W_ae, b_ae,
           W_be, b_be, W_eu1, b_eu1, W_eu2, b_eu2, W_nu1, b_nu1, W_nu2, b_nu2,
           gamma, beta, W_p1, b_p1, W_p2, b_p2, W_p3, b_p3):
    n, fa = x.shape
    e_num, fb = edge_attr.shape
    g = charge.shape[0]
    d = W_ae.shape[1]
    nl = W_eu1.shape[0]
    h_dim = W_p1.shape[1]

    quantum = _NW * _CH * 8
    e_pad = ((e_num + quantum - 1) // quantum) * quantum
    pad = e_pad - e_num
    row = jnp.concatenate([edge_index[0], jnp.zeros((pad,), jnp.int32)])
    col = edge_index[1]
    colg = jnp.concatenate([col, jnp.zeros((pad,), jnp.int32)])
    cols = jnp.concatenate([col, jnp.full((pad,), n, jnp.int32)])
    row2 = row.reshape(e_pad // _CH, _CH)
    colg2 = colg.reshape(e_pad // _CH, _CH)
    cols2 = cols.reshape(e_pad // _CH, _CH)
    ea_pad = jnp.concatenate(
        [edge_attr, jnp.zeros((pad, fb), jnp.float32)], axis=0)

    bn = 2000
    nb = n // bn
    batch_col = batch.reshape(nb, bn, 1)
    batch_row = batch.reshape(1, 1, n)
    zeros2 = jnp.zeros((2, n + 8, d), jnp.float32)

    full = lambda shape: pl.BlockSpec(shape, lambda i: (0,) * len(shape))

    h, h16 = pl.pallas_call(
        _pre_body,
        grid=(nb,),
        in_specs=[
            pl.BlockSpec((bn, fa), lambda i: (i, 0)),
            pl.BlockSpec((1, bn, 1), lambda i: (i, 0, 0)),
            full((g, 1)), full((1, W_ce.shape[1])), full((1, W_ce.shape[1])),
            full((fa, d)), full((W_ce.shape[1], d)), full((1, d)),
        ],
        out_specs=(pl.BlockSpec((bn, d), lambda i: (i, 0)),
                   pl.BlockSpec((bn, d), lambda i: (i, 0))),
        out_shape=(jax.ShapeDtypeStruct((n, d), jnp.float32),
                   jax.ShapeDtypeStruct((n, d), jnp.bfloat16)),
    )(x, batch_col, charge.reshape(g, 1), W_ce, b_ce.reshape(1, -1),
      W_ae[:fa], W_ae[fa:], b_ae.reshape(1, d))

    e_half = e_pad // 2
    ch_half = e_half // _CH
    gather_ab = [_make_sc_gather(n, d // 2, e_half, 0),
                 _make_sc_gather(n, d // 2, e_half, ch_half)]
    scatter_ab = [_make_sc_scatter(n, d, e_half, 0),
                  _make_sc_scatter(n, d, e_half, ch_half)]

    be_blk = 2560
    eb = e_half // be_blk

    def edge_mlp(first, off, xr, xc, e_in, wi):
        w1, b1, w2, b2, wn1, bn1, wn2, bn2 = wi
        f_in = e_in.shape[1]
        ob = off // be_blk
        in_specs = [
            pl.BlockSpec((be_blk, d // 2), lambda i: (i, 0)),
            pl.BlockSpec((be_blk, d // 2), lambda i: (i, 0)),
            pl.BlockSpec((be_blk, f_in),
                         (lambda i: (i + ob, 0)) if first
                         else (lambda i: (i, 0))),
        ]
        args = [xr, xc, e_in]
        if first:
            in_specs += [full((fb, d)), full((1, d))]
            args += [W_be, b_be.reshape(1, d)]
        in_specs += [full((3 * d, d)), full((1, d)), full((d, d)),
                     full((1, d)), full((2 * d, d)), full((1, d)),
                     full((d, d)), full((1, d))]
        args += [w1, b1.reshape(1, d), w2, b2.reshape(1, d),
                 wn1, bn1.reshape(1, d), wn2, bn2.reshape(1, d)]
        return pl.pallas_call(
            _edge_body_first if first else _edge_body,
            grid=(eb,),
            in_specs=in_specs,
            out_specs=(pl.BlockSpec((be_blk, d), lambda i: (i, 0)),
                       pl.BlockSpec((be_blk, d), lambda i: (i, 0))),
            out_shape=(jax.ShapeDtypeStruct((e_half, d), jnp.float32),
                       jax.ShapeDtypeStruct((e_half, d), jnp.float32)),
        )(*args)

    e_h = [ea_pad, ea_pad]
    acc = None
    for i in range(nl):
        hp = _pack_cols(h16)
        wi = (W_eu1[i], b_eu1[i], W_eu2[i], b_eu2[i],
              W_nu1[i], b_nu1[i], W_nu2[i], b_nu2[i])
        gath = [gather_ab[0](hp, row2, colg2),
                gather_ab[1](hp, row2, colg2)]
        msgs = []
        for half in range(2):
            xr, xc = gath[half]
            e_new, msg = edge_mlp(i == 0, half * e_half, xr, xc,
                                  e_h[half], wi)
            e_h[half] = e_new
            msgs.append(msg)
        acc = scatter_ab[0](msgs[0], cols2, zeros2)
        acc = scatter_ab[1](msgs[1], cols2, acc)
        if i < nl - 1:
            h, h16 = pl.pallas_call(
                _update_body,
                grid=(nb,),
                in_specs=[
                    pl.BlockSpec((2, bn, d), lambda i: (0, i, 0)),
                    pl.BlockSpec((bn, d), lambda i: (i, 0)),
                    full((1, d)), full((1, d)),
                ],
                out_specs=(pl.BlockSpec((bn, d), lambda i: (i, 0)),
                           pl.BlockSpec((bn, d), lambda i: (i, 0))),
                out_shape=(jax.ShapeDtypeStruct((n, d), jnp.float32),
                           jax.ShapeDtypeStruct((n, d), jnp.bfloat16)),
            )(acc, h, gamma[i].reshape(1, d), beta[i].reshape(1, d))

    out = pl.pallas_call(
        _final_body,
        grid=(1,),
        in_specs=[
            full((2, n, d)), full((n, d)), full((1, d)), full((1, d)),
            full((1, 1, n)), full((d, h_dim)), full((1, h_dim)),
            full((h_dim, h_dim)), full((1, h_dim)), full((h_dim, 1)),
            full((1, 1)),
        ],
        out_specs=pl.BlockSpec((g, 1), lambda i: (0, 0)),
        out_shape=jax.ShapeDtypeStruct((g, 1), jnp.float32),
    )(acc, h, gamma[nl - 1].reshape(1, d), beta[nl - 1].reshape(1, d),
      batch_row, W_p1, b_p1.reshape(1, h_dim), W_p2, b_p2.reshape(1, h_dim),
      W_p3, b_p3.reshape(1, 1))
    return out[:, 0]


# speedup vs baseline: 2.1993x; 1.0031x over previous
import functools

import jax
import jax.numpy as jnp
from jax import lax
from jax.experimental import pallas as pl
from jax.experimental.pallas import tpu as pltpu
from jax.experimental.pallas import tpu_sc as plsc

_EPS = 1e-5
_SCALE = 1.0 / (1.0 + _EPS) ** 0.5

_CH = 128
_NW = 32


def _softplus(v):
    return jnp.maximum(v, 0.0) + jnp.log1p(jnp.exp(-jnp.abs(v)))


def _pack_cols(h):
    u = jax.lax.bitcast_convert_type(h.astype(jnp.bfloat16), jnp.uint16)
    lo = u[:, :32].astype(jnp.uint32)
    hi = u[:, 32:].astype(jnp.uint32)
    return jax.lax.bitcast_convert_type(lo | (hi << 16), jnp.int32)


def _unpack_cols(v):
    lo = jax.lax.bitcast_convert_type(v << 16, jnp.float32)
    hi = jax.lax.bitcast_convert_type(v & jnp.int32(-65536), jnp.float32)
    return jnp.concatenate([lo, hi], axis=1)


def _pre_body(x_ref, b_ref, ch_ref, wce_ref, bce_ref, waet_ref, waeb_ref,
              bae_ref, h0_ref, h16_ref):
    cf = jnp.dot(ch_ref[...], wce_ref[...],
                 preferred_element_type=jnp.float32) + bce_ref[...]
    q = jnp.dot(cf, waeb_ref[...], preferred_element_type=jnp.float32)
    g = q.shape[0]
    oh = (b_ref[0] == lax.broadcasted_iota(jnp.int32, (1, g), 1)
          ).astype(jnp.float32)
    h0 = (jnp.dot(x_ref[...], waet_ref[...],
                  preferred_element_type=jnp.float32)
          + jnp.dot(oh, q, preferred_element_type=jnp.float32)
          + bae_ref[...])
    h0_ref[...] = h0
    h16_ref[...] = _pack_cols(h0)


def _edge_body_first(xr_ref, xc_ref, ea_ref, wbe_ref, bbe_ref, w1_ref, b1_ref,
                     w2_ref, b2_ref, wn1_ref, bn1_ref, wn2_ref, bn2_ref,
                     enew_ref, msg_ref):
    e = jnp.dot(ea_ref[...], wbe_ref[...],
                preferred_element_type=jnp.float32) + bbe_ref[...]
    _edge_core(_unpack_cols(xr_ref[...]), _unpack_cols(xc_ref[...]), e,
               w1_ref, b1_ref, w2_ref, b2_ref, wn1_ref, bn1_ref, wn2_ref,
               bn2_ref, enew_ref, msg_ref)


def _edge_body(xr_ref, xc_ref, e_ref, w1_ref, b1_ref, w2_ref, b2_ref,
               wn1_ref, bn1_ref, wn2_ref, bn2_ref, enew_ref, msg_ref):
    _edge_core(_unpack_cols(xr_ref[...]), _unpack_cols(xc_ref[...]),
               e_ref[...], w1_ref, b1_ref, w2_ref, b2_ref, wn1_ref, bn1_ref,
               wn2_ref, bn2_ref, enew_ref, msg_ref)


def _edge_core(xr, xc, e, w1_ref, b1_ref, w2_ref, b2_ref, wn1_ref, bn1_ref,
               wn2_ref, bn2_ref, enew_ref, msg_ref):
    ein = jnp.concatenate([xr, xc, e], axis=1)
    t = _softplus(jnp.dot(ein, w1_ref[...],
                          preferred_element_type=jnp.float32) + b1_ref[...])
    enew = jnp.dot(t, w2_ref[...],
                   preferred_element_type=jnp.float32) + b2_ref[...]
    nin = jnp.concatenate([xr, enew], axis=1)
    t2 = _softplus(jnp.dot(nin, wn1_ref[...],
                           preferred_element_type=jnp.float32) + bn1_ref[...])
    msg = jnp.dot(t2, wn2_ref[...],
                  preferred_element_type=jnp.float32) + bn2_ref[...]
    enew_ref[...] = enew
    msg_ref[...] = msg


def _update_body(acc_ref, h_ref, g_ref, be_ref, out_ref, out16_ref):
    s = acc_ref[0] + acc_ref[1]
    xn = s * _SCALE * g_ref[...] + be_ref[...]
    h = _softplus(xn) + h_ref[...]
    out_ref[...] = h
    out16_ref[...] = _pack_cols(h)


def _final_body(acc_ref, h_ref, g_ref, be_ref, b_ref, wp1_ref, bp1_ref,
                wp2_ref, bp2_ref, wp3_ref, bp3_ref, out_ref):
    s = acc_ref[0] + acc_ref[1]
    h = _softplus(s * _SCALE * g_ref[...] + be_ref[...]) + h_ref[...]
    g = out_ref.shape[0]
    oh = (b_ref[0] == lax.broadcasted_iota(jnp.int32, (g, 1), 0)
          ).astype(jnp.float32)
    sums = jnp.dot(oh, h, preferred_element_type=jnp.float32)
    counts = jnp.sum(oh, axis=1, keepdims=True)
    graph = sums / jnp.maximum(counts, 1.0)
    z = _softplus(jnp.dot(graph, wp1_ref[...],
                          preferred_element_type=jnp.float32) + bp1_ref[...])
    z = _softplus(jnp.dot(z, wp2_ref[...],
                          preferred_element_type=jnp.float32) + bp2_ref[...])
    out_ref[...] = jnp.dot(z, wp3_ref[...],
                           preferred_element_type=jnp.float32) + bp3_ref[...]


_NBUF = 8


def _make_sc_gather(n, dw, e_half, cbase):
    cpt = e_half // _CH // _NW
    nbuf = _NBUF
    steps = cpt // nbuf
    mesh = plsc.VectorSubcoreMesh(core_axis_name="c", subcore_axis_name="s")

    def body(h_hbm, row2_hbm, col2_hbm, xr_hbm, xc_hbm, *scr):
        idxr_v, idxc_v, bufr, bufc = scr[:4]
        gs = scr[4:4 + nbuf]
        ws = scr[4 + nbuf:4 + 2 * nbuf]
        c = lax.axis_index("c")
        s = lax.axis_index("s")
        w = s * 2 + c
        start = pl.multiple_of(w * cpt, 8)
        gstart = pl.multiple_of(cbase + w * cpt, 8)
        pltpu.sync_copy(row2_hbm.at[pl.ds(gstart, cpt)], idxr_v)
        pltpu.sync_copy(col2_hbm.at[pl.ds(gstart, cpt)], idxc_v)

        def sstep(S, carry):
            grs = []
            gcs = []
            for b in range(nbuf):
                @pl.when(S > 0)
                def _(b=b):
                    pltpu.make_async_copy(xr_hbm.at[pl.ds(0, _CH)],
                                          bufr.at[b], ws[b]).wait()
                    pltpu.make_async_copy(xc_hbm.at[pl.ds(0, _CH)],
                                          bufc.at[b], ws[b]).wait()
                j = S * nbuf + b
                grs.append(pltpu.async_copy(
                    h_hbm.at[idxr_v.at[j]], bufr.at[b], gs[b]))
                gcs.append(pltpu.async_copy(
                    h_hbm.at[idxc_v.at[j]], bufc.at[b], gs[b]))
            for b in range(nbuf):
                j = S * nbuf + b
                grs[b].wait()
                gcs[b].wait()
                pltpu.async_copy(bufr.at[b],
                                 xr_hbm.at[pl.ds((start + j) * _CH, _CH)],
                                 ws[b])
                pltpu.async_copy(bufc.at[b],
                                 xc_hbm.at[pl.ds((start + j) * _CH, _CH)],
                                 ws[b])
            return carry

        lax.fori_loop(0, steps, sstep, 0)
        for b in range(nbuf):
            pltpu.make_async_copy(xr_hbm.at[pl.ds(0, _CH)], bufr.at[b],
                                  ws[b]).wait()
            pltpu.make_async_copy(xc_hbm.at[pl.ds(0, _CH)], bufc.at[b],
                                  ws[b]).wait()

    return pl.kernel(
        body,
        out_type=(jax.ShapeDtypeStruct((e_half, dw), jnp.int32),
                  jax.ShapeDtypeStruct((e_half, dw), jnp.int32)),
        mesh=mesh,
        compiler_params=pltpu.CompilerParams(use_tc_tiling_on_sc=False),
        scratch_types=[
            pltpu.VMEM((cpt, _CH), jnp.int32),
            pltpu.VMEM((cpt, _CH), jnp.int32),
            pltpu.VMEM((nbuf, _CH, dw), jnp.int32),
            pltpu.VMEM((nbuf, _CH, dw), jnp.int32),
        ] + [pltpu.SemaphoreType.DMA] * (4 * nbuf),
    )


def _make_sc_scatter(n, d, e_half, cbase):
    cpt = e_half // _CH // _NW
    n_acc = n + 8
    rpt_hi = 632
    rpt_lo = n_acc - 15 * rpt_hi
    mesh = plsc.VectorSubcoreMesh(core_axis_name="c", subcore_axis_name="s")

    nbuf = _NBUF
    steps = cpt // nbuf

    def body(msg_hbm, col2_hbm, init_hbm, out_hbm, *scr):
        idx_v, mbuf, accs = scr[:3]
        rsem = scr[3:3 + nbuf]
        ssem = scr[3 + nbuf:3 + 2 * nbuf]
        c = lax.axis_index("c")
        s = lax.axis_index("s")
        w = s * 2 + c

        @pl.when(s == 0)
        def _():
            pltpu.sync_copy(init_hbm.at[c], accs)

        start = pl.multiple_of(w * cpt, 8)
        gstart = pl.multiple_of(cbase + w * cpt, 8)
        pltpu.sync_copy(col2_hbm.at[pl.ds(gstart, cpt)], idx_v)
        plsc.subcore_barrier()

        def sstep(S, carry):
            rds = []
            for b in range(nbuf):
                @pl.when(S > 0)
                def _(b=b):
                    pltpu.make_async_copy(msg_hbm.at[pl.ds(0, _CH)],
                                          mbuf.at[b], ssem[b]).wait()
                j = S * nbuf + b
                rds.append(pltpu.async_copy(
                    msg_hbm.at[pl.ds((start + j) * _CH, _CH)], mbuf.at[b],
                    rsem[b]))
            for b in range(nbuf):
                j = S * nbuf + b
                rds[b].wait()
                pltpu.async_copy(mbuf.at[b], accs.at[idx_v.at[j]], ssem[b],
                                 add=True)
            return carry

        lax.fori_loop(0, steps, sstep, 0)
        for b in range(nbuf):
            pltpu.make_async_copy(msg_hbm.at[pl.ds(0, _CH)], mbuf.at[b],
                                  ssem[b]).wait()
        plsc.subcore_barrier()

        @pl.when(s < 15)
        def _():
            pltpu.sync_copy(accs.at[pl.ds(s * rpt_hi, rpt_hi)],
                            out_hbm.at[c, pl.ds(s * rpt_hi, rpt_hi)])

        @pl.when(s == 15)
        def _():
            pltpu.sync_copy(accs.at[pl.ds(15 * rpt_hi, rpt_lo)],
                            out_hbm.at[c, pl.ds(15 * rpt_hi, rpt_lo)])

    return pl.kernel(
        body,
        out_type=jax.ShapeDtypeStruct((2, n_acc, d), jnp.float32),
        mesh=mesh,
        compiler_params=pltpu.CompilerParams(use_tc_tiling_on_sc=False),
        scratch_types=[
            pltpu.VMEM((cpt, _CH), jnp.int32),
            pltpu.VMEM((_NBUF, _CH, d), jnp.float32),
            pltpu.VMEM_SHARED((n_acc, d), jnp.float32),
        ] + [pltpu.SemaphoreType.DMA] * (2 * _NBUF),
    )


def kernel(x, edge_index, edge_attr, batch, charge, W_ce, b_ce, W_ae, b_ae,
           W_be, b_be, W_eu1, b_eu1, W_eu2, b_eu2, W_nu1, b_nu1, W_nu2, b_nu2,
           gamma, beta, W_p1, b_p1, W_p2, b_p2, W_p3, b_p3):
    n, fa = x.shape
    e_num, fb = edge_attr.shape
    g = charge.shape[0]
    d = W_ae.shape[1]
    nl = W_eu1.shape[0]
    h_dim = W_p1.shape[1]

    quantum = _NW * _CH * 8
    e_pad = ((e_num + quantum - 1) // quantum) * quantum
    pad = e_pad - e_num
    row = jnp.concatenate([edge_index[0], jnp.zeros((pad,), jnp.int32)])
    col = edge_index[1]
    colg = jnp.concatenate([col, jnp.zeros((pad,), jnp.int32)])
    cols = jnp.concatenate([col, jnp.full((pad,), n, jnp.int32)])
    row2 = row.reshape(e_pad // _CH, _CH)
    colg2 = colg.reshape(e_pad // _CH, _CH)
    cols2 = cols.reshape(e_pad // _CH, _CH)
    ea_pad = jnp.concatenate(
        [edge_attr, jnp.zeros((pad, fb), jnp.float32)], axis=0)

    bn = 2000
    nb = n // bn
    batch_col = batch.reshape(nb, bn, 1)
    batch_row = batch.reshape(1, 1, n)
    zeros2 = jnp.zeros((2, n + 8, d), jnp.float32)

    full = lambda shape: pl.BlockSpec(shape, lambda i: (0,) * len(shape))

    h, h16 = pl.pallas_call(
        _pre_body,
        grid=(nb,),
        in_specs=[
            pl.BlockSpec((bn, fa), lambda i: (i, 0)),
            pl.BlockSpec((1, bn, 1), lambda i: (i, 0, 0)),
            full((g, 1)), full((1, W_ce.shape[1])), full((1, W_ce.shape[1])),
            full((fa, d)), full((W_ce.shape[1], d)), full((1, d)),
        ],
        out_specs=(pl.BlockSpec((bn, d), lambda i: (i, 0)),
                   pl.BlockSpec((bn, d // 2), lambda i: (i, 0))),
        out_shape=(jax.ShapeDtypeStruct((n, d), jnp.float32),
                   jax.ShapeDtypeStruct((n, d // 2), jnp.int32)),
    )(x, batch_col, charge.reshape(g, 1), W_ce, b_ce.reshape(1, -1),
      W_ae[:fa], W_ae[fa:], b_ae.reshape(1, d))

    e_half = e_pad // 2
    ch_half = e_half // _CH
    gather_ab = [_make_sc_gather(n, d // 2, e_half, 0),
                 _make_sc_gather(n, d // 2, e_half, ch_half)]
    scatter_ab = [_make_sc_scatter(n, d, e_half, 0),
                  _make_sc_scatter(n, d, e_half, ch_half)]

    be_blk = 2560
    eb = e_half // be_blk

    def edge_mlp(first, off, xr, xc, e_in, wi):
        w1, b1, w2, b2, wn1, bn1, wn2, bn2 = wi
        f_in = e_in.shape[1]
        ob = off // be_blk
        in_specs = [
            pl.BlockSpec((be_blk, d // 2), lambda i: (i, 0)),
            pl.BlockSpec((be_blk, d // 2), lambda i: (i, 0)),
            pl.BlockSpec((be_blk, f_in),
                         (lambda i: (i + ob, 0)) if first
                         else (lambda i: (i, 0))),
        ]
        args = [xr, xc, e_in]
        if first:
            in_specs += [full((fb, d)), full((1, d))]
            args += [W_be, b_be.reshape(1, d)]
        in_specs += [full((3 * d, d)), full((1, d)), full((d, d)),
                     full((1, d)), full((2 * d, d)), full((1, d)),
                     full((d, d)), full((1, d))]
        args += [w1, b1.reshape(1, d), w2, b2.reshape(1, d),
                 wn1, bn1.reshape(1, d), wn2, bn2.reshape(1, d)]
        return pl.pallas_call(
            _edge_body_first if first else _edge_body,
            grid=(eb,),
            in_specs=in_specs,
            out_specs=(pl.BlockSpec((be_blk, d), lambda i: (i, 0)),
                       pl.BlockSpec((be_blk, d), lambda i: (i, 0))),
            out_shape=(jax.ShapeDtypeStruct((e_half, d), jnp.float32),
                       jax.ShapeDtypeStruct((e_half, d), jnp.float32)),
        )(*args)

    e_h = [ea_pad, ea_pad]
    acc = None
    for i in range(nl):
        hp = h16
        wi = (W_eu1[i], b_eu1[i], W_eu2[i], b_eu2[i],
              W_nu1[i], b_nu1[i], W_nu2[i], b_nu2[i])
        gath = [gather_ab[0](hp, row2, colg2),
                gather_ab[1](hp, row2, colg2)]
        msgs = []
        for half in range(2):
            xr, xc = gath[half]
            e_new, msg = edge_mlp(i == 0, half * e_half, xr, xc,
                                  e_h[half], wi)
            e_h[half] = e_new
            msgs.append(msg)
        acc = scatter_ab[0](msgs[0], cols2, zeros2)
        acc = scatter_ab[1](msgs[1], cols2, acc)
        if i < nl - 1:
            h, h16 = pl.pallas_call(
                _update_body,
                grid=(nb,),
                in_specs=[
                    pl.BlockSpec((2, bn, d), lambda i: (0, i, 0)),
                    pl.BlockSpec((bn, d), lambda i: (i, 0)),
                    full((1, d)), full((1, d)),
                ],
                out_specs=(pl.BlockSpec((bn, d), lambda i: (i, 0)),
                           pl.BlockSpec((bn, d // 2), lambda i: (i, 0))),
                out_shape=(jax.ShapeDtypeStruct((n, d), jnp.float32),
                           jax.ShapeDtypeStruct((n, d // 2), jnp.int32)),
            )(acc, h, gamma[i].reshape(1, d), beta[i].reshape(1, d))

    out = pl.pallas_call(
        _final_body,
        grid=(1,),
        in_specs=[
            full((2, n, d)), full((n, d)), full((1, d)), full((1, d)),
            full((1, 1, n)), full((d, h_dim)), full((1, h_dim)),
            full((h_dim, h_dim)), full((1, h_dim)), full((h_dim, 1)),
            full((1, 1)),
        ],
        out_specs=pl.BlockSpec((g, 1), lambda i: (0, 0)),
        out_shape=jax.ShapeDtypeStruct((g, 1), jnp.float32),
    )(acc, h, gamma[nl - 1].reshape(1, d), beta[nl - 1].reshape(1, d),
      batch_row, W_p1, b_p1.reshape(1, h_dim), W_p2, b_p2.reshape(1, h_dim),
      W_p3, b_p3.reshape(1, 1))
    return out[:, 0]
